# baseline (speedup 1.0000x reference)
import functools

import jax
import jax.numpy as jnp
from jax import lax
from jax.experimental import pallas as pl
from jax.experimental.pallas import tpu as pltpu
from jax.experimental.pallas import tpu_sc as plsc

_K = 1024
_C = 64
_B = 8
_HW = 256
_N = _B * _HW
_NC = 2
_NS = 16
_NW = _NC * _NS
_RPW = _N // _NW


def _match_body(x_ref, bank_ref, idx_ref):
    B = x_ref.shape[0]
    hw = x_ref.shape[2]
    bank = bank_ref[...]
    nsq = jnp.sum(bank * bank, axis=1, keepdims=True)
    pn = bank / jnp.maximum(jnp.sqrt(nsq), 1e-12)
    for b in range(B):
        xb = x_ref[b]
        xsq = jnp.sum(xb * xb, axis=0, keepdims=True)
        xn = xb / jnp.maximum(jnp.sqrt(xsq), 1e-12)
        s = jnp.dot(pn, xn, preferred_element_type=jnp.float32)
        idx_ref[b] = jnp.argmax(s, axis=0)[None, :]


_sc_mesh = plsc.VectorSubcoreMesh(core_axis_name="c", subcore_axis_name="s")


@functools.partial(
    pl.kernel,
    mesh=_sc_mesh,
    out_type=jax.ShapeDtypeStruct((_N, 128), jnp.float32),
    scratch_types=[
        pltpu.VMEM((_RPW,), jnp.int32),
        pltpu.VMEM((_RPW, 128), jnp.float32),
        pltpu.SemaphoreType.DMA,
    ],
)
def _sc_gather(table_hbm, idx_hbm, out_hbm, idx_v, rows_v, sem):
    wid = lax.axis_index("s") * _NC + lax.axis_index("c")
    base = wid * _RPW
    pltpu.sync_copy(idx_hbm.at[pl.ds(base, _RPW)], idx_v)
    pltpu.async_copy(table_hbm.at[idx_v], rows_v, sem).wait()
    pltpu.sync_copy(rows_v, out_hbm.at[pl.ds(base, _RPW)])


def kernel(x, prototype_bank):
    B, C, H, W = x.shape
    HW = H * W
    x3 = x.reshape(B, C, HW)
    idx3 = pl.pallas_call(
        _match_body,
        out_shape=jax.ShapeDtypeStruct((B, 1, HW), jnp.int32),
    )(x3, prototype_bank)
    flat_idx = idx3.reshape(B * HW)
    padded = jnp.pad(prototype_bank, ((0, 0), (0, 128 - C)))
    rows = _sc_gather(padded, flat_idx)
    recon = rows[:, :C].reshape(B, H, W, C).transpose(0, 3, 1, 2)
    return recon, idx3.reshape(B, HW)


# speedup vs baseline: 2.2802x; 2.2802x over previous
import jax
import jax.numpy as jnp
from jax.experimental import pallas as pl

_K = 1024
_C = 64


def _match_body(x_ref, bank_ref, recon_ref, idx_ref):
    B = x_ref.shape[0]
    hw = x_ref.shape[2]
    bank = bank_ref[...]
    nsq = jnp.sum(bank * bank, axis=1, keepdims=True)
    pn = bank / jnp.maximum(jnp.sqrt(nsq), 1e-12)
    cols = []
    for b in range(B):
        xb = x_ref[b]
        xsq = jnp.sum(xb * xb, axis=0, keepdims=True)
        cols.append(xb / jnp.maximum(jnp.sqrt(xsq), 1e-12))
    xn = jnp.concatenate(cols, axis=1)
    s = jnp.dot(pn, xn, preferred_element_type=jnp.float32)
    idx = jnp.argmax(s, axis=0)[None, :]
    iota = jax.lax.broadcasted_iota(jnp.int32, (_K, B * hw), 0)
    onehot = (iota == idx).astype(jnp.float32)
    recon = jax.lax.dot_general(
        bank, onehot, (((0,), (0,)), ((), ())),
        preferred_element_type=jnp.float32)
    for b in range(B):
        recon_ref[b] = recon[:, b * hw:(b + 1) * hw]
        idx_ref[b] = idx[:, b * hw:(b + 1) * hw]


def kernel(x, prototype_bank):
    B, C, H, W = x.shape
    HW = H * W
    x3 = x.reshape(B, C, HW)
    recon3, idx3 = pl.pallas_call(
        _match_body,
        out_shape=[
            jax.ShapeDtypeStruct((B, C, HW), jnp.float32),
            jax.ShapeDtypeStruct((B, 1, HW), jnp.int32),
        ],
    )(x3, prototype_bank)
    return recon3.reshape(B, C, H, W), idx3.reshape(B, HW)
